# baseline (speedup 1.0000x reference)
import functools

import jax
import jax.numpy as jnp
from jax import lax
from jax.experimental import pallas as pl
from jax.experimental.pallas import tpu as pltpu
from jax.experimental.pallas import tpu_sc as plsc

BATCH = 16384
EMB = 64
LANES = 16
NUM_CORES = 2
NUM_SUBCORES = 16
NW = NUM_CORES * NUM_SUBCORES
ROWS_PER_W = BATCH // NW
GATHER_CHUNK = 128
NCHUNK = ROWS_PER_W // GATHER_CHUNK
GROUPS = ROWS_PER_W // LANES


def _sc_body(h_ref, t_ref, r_ref, ent_ref, rel_ref, out_ref,
             h_idx, t_idx, r_idx, he, te, re, out_v,
             sem_h, sem_t, sem_r):
    wid = lax.axis_index("s") * NUM_CORES + lax.axis_index("c")
    base = wid * ROWS_PER_W

    pltpu.sync_copy(h_ref.at[pl.ds(base, ROWS_PER_W)], h_idx)
    pltpu.sync_copy(t_ref.at[pl.ds(base, ROWS_PER_W)], t_idx)
    pltpu.sync_copy(r_ref.at[pl.ds(base, ROWS_PER_W)], r_idx)

    copies = []
    for j in range(NCHUNK):
        sl = pl.ds(j * GATHER_CHUNK, GATHER_CHUNK)
        copies.append(pltpu.async_copy(
            ent_ref.at[h_idx.at[sl]], he.at[sl, :], sem_h))
        copies.append(pltpu.async_copy(
            ent_ref.at[t_idx.at[sl]], te.at[sl, :], sem_t))
        copies.append(pltpu.async_copy(
            rel_ref.at[r_idx.at[sl]], re.at[sl, :], sem_r))
    for c in copies:
        c.wait()

    lane = lax.iota(jnp.int32, LANES)
    rots = [(lane + sh) % LANES for sh in (8, 4, 2, 1)]

    def group_body(g, _):
        acc = jnp.zeros((LANES,), jnp.float32)
        for i in range(LANES):
            row = g * LANES + i
            c = None
            for k in range(EMB // LANES):
                sl = pl.ds(k * LANES, LANES)
                d = jnp.abs(he[row, sl] + re[row, sl] - te[row, sl])
                c = d if c is None else c + d
            for ridx in rots:
                c = c + c[ridx]
            acc = jnp.where(lane == i, c, acc)
        out_v[pl.ds(g * LANES, LANES)] = acc
        return _

    lax.fori_loop(0, GROUPS, group_body, None)

    pltpu.sync_copy(out_v, out_ref.at[pl.ds(base, ROWS_PER_W)])


@functools.partial(jax.jit, static_argnums=())
def kernel(h, t, r, ent_weight, rel_weight):
    mesh = plsc.VectorSubcoreMesh(core_axis_name="c", subcore_axis_name="s")
    f = pl.kernel(
        _sc_body,
        out_type=jax.ShapeDtypeStruct((BATCH,), jnp.float32),
        mesh=mesh,
        compiler_params=pltpu.CompilerParams(use_tc_tiling_on_sc=False),
        scratch_types=[
            pltpu.VMEM((ROWS_PER_W,), jnp.int32),
            pltpu.VMEM((ROWS_PER_W,), jnp.int32),
            pltpu.VMEM((ROWS_PER_W,), jnp.int32),
            pltpu.VMEM((ROWS_PER_W, EMB), jnp.float32),
            pltpu.VMEM((ROWS_PER_W, EMB), jnp.float32),
            pltpu.VMEM((ROWS_PER_W, EMB), jnp.float32),
            pltpu.VMEM((ROWS_PER_W,), jnp.float32),
            pltpu.SemaphoreType.DMA,
            pltpu.SemaphoreType.DMA,
            pltpu.SemaphoreType.DMA,
        ],
    )
    return f(h, t, r, ent_weight, rel_weight)


# speedup vs baseline: 1.1031x; 1.1031x over previous
import functools

import jax
import jax.numpy as jnp
from jax import lax
from jax.experimental import pallas as pl
from jax.experimental.pallas import tpu as pltpu
from jax.experimental.pallas import tpu_sc as plsc

BATCH = 16384
EMB = 64
PADDED = 128
LANES = 16
NUM_CORES = 2
NUM_SUBCORES = 16
NW = NUM_CORES * NUM_SUBCORES
ROWS_PER_W = BATCH // NW
GATHER_CHUNK = 128
PHASE_ROWS = 256
NPHASE = ROWS_PER_W // PHASE_ROWS
GROUPS = PHASE_ROWS // LANES


def _sc_body(h_ref, t_ref, r_ref, ent_ref, rel_ref, out_ref,
             h_idx, t_idx, r_idx, he, te, re, out_v,
             sem_h, sem_t, sem_r):
    wid = lax.axis_index("s") * NUM_CORES + lax.axis_index("c")
    base = wid * ROWS_PER_W

    pltpu.sync_copy(h_ref.at[pl.ds(base, ROWS_PER_W)], h_idx)
    pltpu.sync_copy(t_ref.at[pl.ds(base, ROWS_PER_W)], t_idx)
    pltpu.sync_copy(r_ref.at[pl.ds(base, ROWS_PER_W)], r_idx)

    lane = lax.iota(jnp.int32, LANES)
    rots = [(lane + sh) % LANES for sh in (8, 4, 2, 1)]

    for p in range(NPHASE):
        copies = []
        for j in range(PHASE_ROWS // GATHER_CHUNK):
            src = pl.ds(p * PHASE_ROWS + j * GATHER_CHUNK, GATHER_CHUNK)
            dst = pl.ds(j * GATHER_CHUNK, GATHER_CHUNK)
            copies.append(pltpu.async_copy(
                ent_ref.at[h_idx.at[src]], he.at[dst, :], sem_h))
            copies.append(pltpu.async_copy(
                ent_ref.at[t_idx.at[src]], te.at[dst, :], sem_t))
            copies.append(pltpu.async_copy(
                rel_ref.at[r_idx.at[src]], re.at[dst, :], sem_r))
        for c in copies:
            c.wait()

        def group_body(g, _, _p=p):
            acc = jnp.zeros((LANES,), jnp.float32)
            for i in range(LANES):
                row = g * LANES + i
                c = None
                for k in range(EMB // LANES):
                    sl = pl.ds(k * LANES, LANES)
                    d = jnp.abs(he[row, sl] + re[row, sl] - te[row, sl])
                    c = d if c is None else c + d
                for ridx in rots:
                    c = c + c[ridx]
                acc = jnp.where(lane == i, c, acc)
            out_v[pl.ds(_p * PHASE_ROWS + g * LANES, LANES)] = acc
            return _

        lax.fori_loop(0, GROUPS, group_body, None)

    pltpu.sync_copy(out_v, out_ref.at[pl.ds(base, ROWS_PER_W)])


@functools.partial(jax.jit, static_argnums=())
def kernel(h, t, r, ent_weight, rel_weight):
    ent_p = jnp.pad(ent_weight, ((0, 0), (0, PADDED - EMB)))
    rel_p = jnp.pad(rel_weight, ((0, 0), (0, PADDED - EMB)))
    mesh = plsc.VectorSubcoreMesh(core_axis_name="c", subcore_axis_name="s")
    f = pl.kernel(
        _sc_body,
        out_type=jax.ShapeDtypeStruct((BATCH,), jnp.float32),
        mesh=mesh,
        scratch_types=[
            pltpu.VMEM((ROWS_PER_W,), jnp.int32),
            pltpu.VMEM((ROWS_PER_W,), jnp.int32),
            pltpu.VMEM((ROWS_PER_W,), jnp.int32),
            pltpu.VMEM((PHASE_ROWS, PADDED), jnp.float32),
            pltpu.VMEM((PHASE_ROWS, PADDED), jnp.float32),
            pltpu.VMEM((PHASE_ROWS, PADDED), jnp.float32),
            pltpu.VMEM((ROWS_PER_W,), jnp.float32),
            pltpu.SemaphoreType.DMA,
            pltpu.SemaphoreType.DMA,
            pltpu.SemaphoreType.DMA,
        ],
    )
    return f(h, t, r, ent_p, rel_p)


# speedup vs baseline: 1.9622x; 1.7788x over previous
import functools

import jax
import jax.numpy as jnp
from jax import lax
from jax.experimental import pallas as pl
from jax.experimental.pallas import tpu as pltpu
from jax.experimental.pallas import tpu_sc as plsc

BATCH = 16384
ENT_ROWS = 1000000
REL_ROWS = 1000
EMB = 64
LANES = 16
NUM_CORES = 2
NUM_SUBCORES = 16
NW = NUM_CORES * NUM_SUBCORES
ROWS_PER_W = BATCH // NW
PH = 32
NPHASE = ROWS_PER_W // PH
CHUNKS = PH // LANES


def _sc_body(h_ref, t_ref, r_ref, ent_ref, rel_ref, out_ref,
             h_idx, t_idx, r_idx, he8, te8, re8, out_v, sem):
    wid = lax.axis_index("s") * NUM_CORES + lax.axis_index("c")
    base = wid * ROWS_PER_W

    pltpu.sync_copy(h_ref.at[pl.ds(base, ROWS_PER_W)], h_idx)
    pltpu.sync_copy(t_ref.at[pl.ds(base, ROWS_PER_W)], t_idx)
    pltpu.sync_copy(r_ref.at[pl.ds(base, ROWS_PER_W)], r_idx)

    lane = lax.iota(jnp.int32, LANES)
    rots = [(lane + sh) % LANES for sh in (8, 4, 2, 1)]

    def phase_body(p, _):
        tbl_es = []
        for idx_v, tab, buf in ((h_idx, ent_ref, he8),
                                (t_idx, ent_ref, te8),
                                (r_idx, rel_ref, re8)):
            es = []
            for c in range(CHUNKS):
                vec = idx_v[pl.ds(p * PH + c * LANES, LANES)]
                es.extend(vec[j] for j in range(LANES))
            tbl_es.append(es)
            for el in range(PH):
                pltpu.async_copy(tab.at[es[el] >> 3], buf.at[el], sem)
        es_h, es_t, es_r = tbl_es

        pltpu.make_async_copy(ent_ref.at[pl.ds(0, PH)], he8, sem).wait()
        pltpu.make_async_copy(ent_ref.at[pl.ds(0, PH)], te8, sem).wait()
        pltpu.make_async_copy(ent_ref.at[pl.ds(0, PH)], re8, sem).wait()

        for g in range(CHUNKS):
            acc = jnp.zeros((LANES,), jnp.float32)
            for i in range(LANES):
                el = g * LANES + i
                hm = es_h[el] & 7
                tm = es_t[el] & 7
                rm = es_r[el] & 7
                c = None
                for k in range(EMB // LANES):
                    sl = pl.ds(k * LANES, LANES)
                    d = jnp.abs(he8[el, hm, sl] + re8[el, rm, sl]
                                - te8[el, tm, sl])
                    c = d if c is None else c + d
                for ridx in rots:
                    c = c + c[ridx]
                acc = jnp.where(lane == i, c, acc)
            out_v[pl.ds(p * PH + g * LANES, LANES)] = acc
        return _

    lax.fori_loop(0, NPHASE, phase_body, None)

    pltpu.sync_copy(out_v, out_ref.at[pl.ds(base, ROWS_PER_W)])


@functools.partial(jax.jit, static_argnums=())
def kernel(h, t, r, ent_weight, rel_weight):
    ent3 = ent_weight.reshape(ENT_ROWS // 8, 8, EMB)
    rel3 = rel_weight.reshape(REL_ROWS // 8, 8, EMB)
    mesh = plsc.VectorSubcoreMesh(core_axis_name="c", subcore_axis_name="s")
    f = pl.kernel(
        _sc_body,
        out_type=jax.ShapeDtypeStruct((BATCH,), jnp.float32),
        mesh=mesh,
        scratch_types=[
            pltpu.VMEM((ROWS_PER_W,), jnp.int32),
            pltpu.VMEM((ROWS_PER_W,), jnp.int32),
            pltpu.VMEM((ROWS_PER_W,), jnp.int32),
            pltpu.VMEM((PH, 8, EMB), jnp.float32),
            pltpu.VMEM((PH, 8, EMB), jnp.float32),
            pltpu.VMEM((PH, 8, EMB), jnp.float32),
            pltpu.VMEM((ROWS_PER_W,), jnp.float32),
            pltpu.SemaphoreType.DMA,
        ],
    )
    return f(h, t, r, ent3, rel3)
